# baseline (speedup 1.0000x reference)
import functools

import jax
import jax.numpy as jnp
from jax import lax
from jax.experimental import pallas as pl
from jax.experimental.pallas import tpu as pltpu
from jax.experimental.pallas import tpu_sc as plsc

N_NODES = 10000
D_FEAT = 128
N_EDGES = 320000
BATCH = 4096
H = D_FEAT // 2
NC, NS, L = 2, 16, 16
EPT = N_EDGES // NS
KB = 128
EBUF = EPT + 784
ACC_ROWS = BATCH + L
DUMP = BATCH
ZROWS = ACC_ROWS // NS
RPT = BATCH // NS
XRT = N_NODES // NS


@functools.partial(
    pl.kernel,
    out_type=jax.ShapeDtypeStruct((2, BATCH, H), jnp.float32),
    mesh=plsc.VectorSubcoreMesh(core_axis_name="c", subcore_axis_name="s"),
    compiler_params=pltpu.CompilerParams(use_tc_tiling_on_sc=False,
                                         needs_layout_passes=False),
    scratch_types=(
        [
            pltpu.VMEM((EBUF,), jnp.int32),
            pltpu.VMEM((EBUF,), jnp.int32),
        ] +
        [pltpu.VMEM((KB, H), jnp.float32) for _ in range(4)] +
        [
            pltpu.VMEM((KB,), jnp.int32),
            pltpu.VMEM_SHARED((ACC_ROWS, H), jnp.float32),
            pltpu.VMEM_SHARED((N_NODES, H), jnp.float32),
        ] +
        [pltpu.SemaphoreType.DMA for _ in range(8)]
    ),
)
def _agg_kernel(x2, src, dstp, zrows, out,
                sbuf, dbuf,
                rows0, rows1, rows2, rows3,
                tidx, acc, xsp,
                gsem0, gsem1, gsem2, gsem3,
                ssem0, ssem1, ssem2, ssem3):
    c = lax.axis_index("c")
    s = lax.axis_index("s")
    rows = (rows0, rows1, rows2, rows3)
    gsem = (gsem0, gsem1, gsem2, gsem3)
    ssem = (ssem0, ssem1, ssem2, ssem3)

    pltpu.sync_copy(zrows, acc.at[pl.ds(s * ZROWS, ZROWS)])
    pltpu.sync_copy(src.at[pl.ds(s * EPT, EPT)], sbuf.at[pl.ds(0, EPT)])
    pltpu.sync_copy(dstp.at[pl.ds(s * EPT, EPT)], dbuf.at[pl.ds(0, EPT)])

    iota = lax.iota(jnp.int32, L)
    tbase = s * XRT

    def build_tidx(b):
        for k in range(KB // L):
            r = jnp.minimum(tbase + b * KB + k * L + iota, tbase + XRT - 1)
            tidx[pl.ds(k * L, L)] = r * 2 + c

    build_tidx(0)
    pltpu.async_copy(x2.at[tidx], rows[0], gsem[0])
    for b in range(5):
        n = KB if b < 4 else XRT - 4 * KB
        pltpu.make_async_copy(x2.at[tidx], rows[b % 2], gsem[b % 2]).wait()
        pltpu.sync_copy(rows[b % 2].at[pl.ds(0, n)],
                        xsp.at[pl.ds(tbase + b * KB, n)])
        if b + 1 < 5:
            build_tidx(b + 1)
            pltpu.async_copy(x2.at[tidx], rows[(b + 1) % 2],
                             gsem[(b + 1) % 2])

    plsc.subcore_barrier()

    def pack_sort(i):
        svv = sbuf[pl.ds(i * L, L)]
        dvv = dbuf[pl.ds(i * L, L)]
        keep = (dvv >= 0) & (dvv < BATCH)
        dcl = jnp.where(keep, dvv, DUMP)
        key = (jnp.where(keep, 0, 1 << 27) + (svv << 13) + dcl)
        return jnp.sort(key), jnp.sum(keep.astype(jnp.int32))

    def unpack_store(ks, off):
        sbuf[pl.ds(off, L)] = (ks >> 13) & 0x3FFF
        dbuf[pl.ds(off, L)] = ks & 0x1FFF

    def comp_body(i, off):
        ks_a, cnt_a = pack_sort(4 * i)
        ks_b, cnt_b = pack_sort(4 * i + 1)
        ks_c, cnt_c = pack_sort(4 * i + 2)
        ks_d, cnt_d = pack_sort(4 * i + 3)
        unpack_store(ks_a, off)
        off_b = off + cnt_a
        unpack_store(ks_b, off_b)
        off_c = off_b + cnt_b
        unpack_store(ks_c, off_c)
        off_d = off_c + cnt_c
        unpack_store(ks_d, off_d)
        return off_d + cnt_d

    n_keep = lax.fori_loop(0, EPT // (4 * L), comp_body, jnp.int32(0))
    ks_a, cnt_a = pack_sort(jnp.int32(EPT // L - 2))
    ks_b, cnt_b = pack_sort(jnp.int32(EPT // L - 1))
    unpack_store(ks_a, n_keep)
    unpack_store(ks_b, n_keep + cnt_a)
    n_keep = n_keep + cnt_a + cnt_b

    zvec = jnp.zeros((L,), jnp.int32)
    dumpv = jnp.full((L,), DUMP, jnp.int32)

    def pad_body(j, _):
        sbuf[pl.ds(n_keep + j * L, L)] = zvec
        dbuf[pl.ds(n_keep + j * L, L)] = dumpv
        return 0

    lax.fori_loop(0, 40, pad_body, 0)
    nb = (n_keep + KB - 1) // KB
    nb4 = 4 * jnp.maximum(1, (nb + 3) // 4)

    def start_gather(t, p):
        pltpu.async_copy(xsp.at[sbuf.at[pl.ds(t * KB, KB)]], rows[p], gsem[p])

    def wait_gather(p):
        pltpu.make_async_copy(xsp.at[sbuf.at[pl.ds(0, KB)]], rows[p],
                              gsem[p]).wait()

    def start_scatter(t, p):
        pltpu.async_copy(rows[p], acc.at[dbuf.at[pl.ds(t * KB, KB)]],
                         ssem[p], add=True)

    def wait_scatter(p):
        pltpu.make_async_copy(rows[p], acc.at[dbuf.at[pl.ds(0, KB)]],
                              ssem[p]).wait()

    start_gather(jnp.int32(0), 0)
    start_gather(jnp.int32(1), 1)
    for t in range(2, 4):
        start_gather(jnp.int32(t), t)
        wait_gather(t - 2)
        start_scatter(jnp.int32(t - 2), t - 2)

    def group_body(g, _):
        for p in range(4):
            t = 4 * g + p
            wait_scatter(p)
            start_gather(t, p)
            wait_gather((p + 2) % 4)
            start_scatter(t - 2, (p + 2) % 4)
        return 0

    lax.fori_loop(1, nb4 // 4, group_body, 0)

    wait_gather(2)
    start_scatter(nb4 - 2, 2)
    wait_gather(3)
    start_scatter(nb4 - 1, 3)
    for p in range(4):
        wait_scatter(p)

    plsc.subcore_barrier()

    base = s * RPT
    pltpu.sync_copy(acc.at[pl.ds(base, RPT)],
                    out.at[c, pl.ds(base, RPT), :])


def kernel(x, edge_index, batch_size):
    x = x.astype(jnp.float32)
    ei = edge_index.astype(jnp.int32)
    start = jnp.asarray(batch_size, jnp.int32) - BATCH
    src = ei[0]
    dstp = ei[1] - start
    x2 = x.reshape(2 * N_NODES, H)
    zrows = jnp.zeros((ZROWS, H), jnp.float32)
    planes = _agg_kernel(x2, src, dstp, zrows)
    sums = planes.transpose(1, 0, 2).reshape(BATCH, D_FEAT)
    x_head = lax.dynamic_slice_in_dim(x, start, BATCH, axis=0)
    return jnp.concatenate((x_head, sums), axis=-1)


# speedup vs baseline: 1.0377x; 1.0377x over previous
import functools

import jax
import jax.numpy as jnp
from jax import lax
from jax.experimental import pallas as pl
from jax.experimental.pallas import tpu as pltpu
from jax.experimental.pallas import tpu_sc as plsc

N_NODES = 10000
D_FEAT = 128
N_EDGES = 320000
BATCH = 4096
H = D_FEAT // 2
NC, NS, L = 2, 16, 16
EPT = N_EDGES // NS
KB = 128
EBUF = EPT + 784
ACC_ROWS = BATCH + L
DUMP = BATCH
ZROWS = ACC_ROWS // NS
RPT = BATCH // NS
XRT = N_NODES // NS


@functools.partial(
    pl.kernel,
    out_type=jax.ShapeDtypeStruct((2, BATCH, H), jnp.float32),
    mesh=plsc.VectorSubcoreMesh(core_axis_name="c", subcore_axis_name="s"),
    compiler_params=pltpu.CompilerParams(use_tc_tiling_on_sc=False,
                                         needs_layout_passes=False),
    scratch_types=(
        [
            pltpu.VMEM((EBUF,), jnp.int32),
            pltpu.VMEM((EBUF,), jnp.int32),
        ] +
        [pltpu.VMEM((KB, H), jnp.float32) for _ in range(4)] +
        [
            pltpu.VMEM((KB,), jnp.int32),
            pltpu.VMEM_SHARED((ACC_ROWS, H), jnp.float32),
            pltpu.VMEM_SHARED((N_NODES, H), jnp.float32),
        ] +
        [pltpu.SemaphoreType.DMA for _ in range(8)]
    ),
)
def _agg_kernel(x2, src, dstp, zrows, out,
                sbuf, dbuf,
                rows0, rows1, rows2, rows3,
                tidx, acc, xsp,
                gsem0, gsem1, gsem2, gsem3,
                ssem0, ssem1, ssem2, ssem3):
    c = lax.axis_index("c")
    s = lax.axis_index("s")
    rows = (rows0, rows1, rows2, rows3)
    gsem = (gsem0, gsem1, gsem2, gsem3)
    ssem = (ssem0, ssem1, ssem2, ssem3)

    pltpu.sync_copy(zrows, acc.at[pl.ds(s * ZROWS, ZROWS)])
    pltpu.sync_copy(src.at[pl.ds(s * EPT, EPT)], sbuf.at[pl.ds(0, EPT)])
    pltpu.sync_copy(dstp.at[pl.ds(s * EPT, EPT)], dbuf.at[pl.ds(0, EPT)])

    iota = lax.iota(jnp.int32, L)
    tbase = s * XRT

    def build_tidx(b):
        for k in range(KB // L):
            r = jnp.minimum(tbase + b * KB + k * L + iota, tbase + XRT - 1)
            tidx[pl.ds(k * L, L)] = r * 2 + c

    build_tidx(0)
    pltpu.async_copy(x2.at[tidx], rows[0], gsem[0])
    for b in range(5):
        n = KB if b < 4 else XRT - 4 * KB
        pltpu.make_async_copy(x2.at[tidx], rows[b % 2], gsem[b % 2]).wait()
        pltpu.sync_copy(rows[b % 2].at[pl.ds(0, n)],
                        xsp.at[pl.ds(tbase + b * KB, n)])
        if b + 1 < 5:
            build_tidx(b + 1)
            pltpu.async_copy(x2.at[tidx], rows[(b + 1) % 2],
                             gsem[(b + 1) % 2])

    plsc.subcore_barrier()

    def pack_sort(i):
        svv = sbuf[pl.ds(i * L, L)]
        dvv = dbuf[pl.ds(i * L, L)]
        keep = (dvv >= 0) & (dvv < BATCH)
        dcl = jnp.where(keep, dvv, DUMP)
        key = (jnp.where(keep, 0, 1 << 27) + (svv << 13) + dcl)
        return jnp.sort(key), jnp.sum(keep.astype(jnp.int32))

    def unpack_store(ks, off):
        sbuf[pl.ds(off, L)] = (ks >> 13) & 0x3FFF
        dbuf[pl.ds(off, L)] = ks & 0x1FFF

    def comp_body(i, off):
        ks_a, cnt_a = pack_sort(4 * i)
        ks_b, cnt_b = pack_sort(4 * i + 1)
        ks_c, cnt_c = pack_sort(4 * i + 2)
        ks_d, cnt_d = pack_sort(4 * i + 3)
        unpack_store(ks_a, off)
        off_b = off + cnt_a
        unpack_store(ks_b, off_b)
        off_c = off_b + cnt_b
        unpack_store(ks_c, off_c)
        off_d = off_c + cnt_c
        unpack_store(ks_d, off_d)
        return off_d + cnt_d

    n_keep = lax.fori_loop(0, EPT // (4 * L), comp_body, jnp.int32(0))
    ks_a, cnt_a = pack_sort(jnp.int32(EPT // L - 2))
    ks_b, cnt_b = pack_sort(jnp.int32(EPT // L - 1))
    unpack_store(ks_a, n_keep)
    unpack_store(ks_b, n_keep + cnt_a)
    n_keep = n_keep + cnt_a + cnt_b

    zvec = jnp.zeros((L,), jnp.int32)
    dumpv = jnp.full((L,), DUMP, jnp.int32)

    def pad_body(j, _):
        sbuf[pl.ds(n_keep + j * L, L)] = zvec
        dbuf[pl.ds(n_keep + j * L, L)] = dumpv
        return 0

    lax.fori_loop(0, 40, pad_body, 0)
    nb = (n_keep + KB - 1) // KB
    nb4 = 4 * jnp.maximum(1, (nb + 3) // 4)

    def start_gather(t, p):
        pltpu.async_copy(xsp.at[sbuf.at[pl.ds(t * KB, KB)]], rows[p], gsem[p])

    def wait_gather(p):
        pltpu.make_async_copy(xsp.at[sbuf.at[pl.ds(0, KB)]], rows[p],
                              gsem[p]).wait()

    def start_scatter(t, p):
        pltpu.async_copy(rows[p], acc.at[dbuf.at[pl.ds(t * KB, KB)]],
                         ssem[p], add=True)

    def wait_scatter(p):
        pltpu.make_async_copy(rows[p], acc.at[dbuf.at[pl.ds(0, KB)]],
                              ssem[p]).wait()

    start_gather(jnp.int32(0), 0)
    start_gather(jnp.int32(1), 1)
    for t in range(2, 4):
        start_gather(jnp.int32(t), t)
        wait_gather(t - 2)
        start_scatter(jnp.int32(t - 2), t - 2)

    def group_body(g, _):
        for p in range(4):
            t = 4 * g + p
            wait_scatter(p)
            start_gather(t, p)
            wait_gather((p + 2) % 4)
            start_scatter(t - 2, (p + 2) % 4)
        return 0

    lax.fori_loop(1, nb4 // 4, group_body, 0)

    wait_gather(2)
    start_scatter(nb4 - 2, 2)
    wait_gather(3)
    start_scatter(nb4 - 1, 3)
    for p in range(4):
        wait_scatter(p)

    plsc.subcore_barrier()

    base = s * RPT
    pltpu.sync_copy(acc.at[pl.ds(base, RPT)],
                    out.at[c, pl.ds(base, RPT), :])


def kernel(x, edge_index, batch_size):
    x = x.astype(jnp.float32)
    ei = edge_index.astype(jnp.int32)
    start = jnp.asarray(batch_size, jnp.int32) - BATCH
    src = ei[0]
    dstp = ei[1] - start
    x2 = x.reshape(2 * N_NODES, H)
    zrows = jnp.zeros((ZROWS, H), jnp.float32)
    planes = _agg_kernel(x2, src, dstp, zrows)
    x_head = lax.dynamic_slice_in_dim(x, start, BATCH, axis=0)
    return jnp.concatenate((x_head, planes[0], planes[1]), axis=-1)


# speedup vs baseline: 1.1506x; 1.1088x over previous
import functools

import jax
import jax.numpy as jnp
from jax import lax
from jax.experimental import pallas as pl
from jax.experimental.pallas import tpu as pltpu
from jax.experimental.pallas import tpu_sc as plsc

N_NODES = 10000
D_FEAT = 128
N_EDGES = 320000
BATCH = 4096
H = D_FEAT // 2
NC, NS, L = 2, 16, 16
EPT = N_EDGES // NS
KB = 128
EBUF = EPT + 784
ACC_ROWS = BATCH + L
DUMP = BATCH
ZROWS = ACC_ROWS // NS
RPT = BATCH // NS
XRT = N_NODES // NS


@functools.partial(
    pl.kernel,
    out_type=jax.ShapeDtypeStruct((2, BATCH, H), jnp.float32),
    mesh=plsc.VectorSubcoreMesh(core_axis_name="c", subcore_axis_name="s"),
    compiler_params=pltpu.CompilerParams(use_tc_tiling_on_sc=False,
                                         needs_layout_passes=False),
    scratch_types=(
        [
            pltpu.VMEM((EBUF,), jnp.int32),
            pltpu.VMEM((EBUF,), jnp.int32),
        ] +
        [pltpu.VMEM((KB, H), jnp.float32) for _ in range(4)] +
        [
            pltpu.VMEM((KB,), jnp.int32),
            pltpu.VMEM_SHARED((ACC_ROWS, H), jnp.float32),
            pltpu.VMEM_SHARED((N_NODES, H), jnp.float32),
        ] +
        [pltpu.SemaphoreType.DMA for _ in range(8)]
    ),
)
def _agg_kernel(x2, ei, startv, zrows, out,
                sbuf, dbuf,
                rows0, rows1, rows2, rows3,
                tidx, acc, xsp,
                gsem0, gsem1, gsem2, gsem3,
                ssem0, ssem1, ssem2, ssem3):
    c = lax.axis_index("c")
    s = lax.axis_index("s")
    rows = (rows0, rows1, rows2, rows3)
    gsem = (gsem0, gsem1, gsem2, gsem3)
    ssem = (ssem0, ssem1, ssem2, ssem3)

    pltpu.sync_copy(zrows, acc.at[pl.ds(s * ZROWS, ZROWS)])
    pltpu.sync_copy(ei.at[0, pl.ds(s * EPT, EPT)], sbuf.at[pl.ds(0, EPT)])
    pltpu.sync_copy(ei.at[1, pl.ds(s * EPT, EPT)], dbuf.at[pl.ds(0, EPT)])
    pltpu.sync_copy(startv, tidx.at[pl.ds(0, L)])
    start_vec = tidx[pl.ds(0, L)]

    iota = lax.iota(jnp.int32, L)
    tbase = s * XRT

    def build_tidx(b):
        for k in range(KB // L):
            r = jnp.minimum(tbase + b * KB + k * L + iota, tbase + XRT - 1)
            tidx[pl.ds(k * L, L)] = r * 2 + c

    build_tidx(0)
    pltpu.async_copy(x2.at[tidx], rows[0], gsem[0])
    for b in range(5):
        n = KB if b < 4 else XRT - 4 * KB
        pltpu.make_async_copy(x2.at[tidx], rows[b % 2], gsem[b % 2]).wait()
        pltpu.sync_copy(rows[b % 2].at[pl.ds(0, n)],
                        xsp.at[pl.ds(tbase + b * KB, n)])
        if b + 1 < 5:
            build_tidx(b + 1)
            pltpu.async_copy(x2.at[tidx], rows[(b + 1) % 2],
                             gsem[(b + 1) % 2])

    plsc.subcore_barrier()

    def pack_sort(i):
        svv = sbuf[pl.ds(i * L, L)]
        dvv = dbuf[pl.ds(i * L, L)] - start_vec
        keep = (dvv >= 0) & (dvv < BATCH)
        dcl = jnp.where(keep, dvv, DUMP)
        key = (jnp.where(keep, 0, 1 << 27) + (svv << 13) + dcl)
        return jnp.sort(key), jnp.sum(keep.astype(jnp.int32))

    def unpack_store(ks, off):
        sbuf[pl.ds(off, L)] = (ks >> 13) & 0x3FFF
        dbuf[pl.ds(off, L)] = ks & 0x1FFF

    def comp_body(i, off):
        ks_a, cnt_a = pack_sort(4 * i)
        ks_b, cnt_b = pack_sort(4 * i + 1)
        ks_c, cnt_c = pack_sort(4 * i + 2)
        ks_d, cnt_d = pack_sort(4 * i + 3)
        unpack_store(ks_a, off)
        off_b = off + cnt_a
        unpack_store(ks_b, off_b)
        off_c = off_b + cnt_b
        unpack_store(ks_c, off_c)
        off_d = off_c + cnt_c
        unpack_store(ks_d, off_d)
        return off_d + cnt_d

    n_keep = lax.fori_loop(0, EPT // (4 * L), comp_body, jnp.int32(0))
    ks_a, cnt_a = pack_sort(jnp.int32(EPT // L - 2))
    ks_b, cnt_b = pack_sort(jnp.int32(EPT // L - 1))
    unpack_store(ks_a, n_keep)
    unpack_store(ks_b, n_keep + cnt_a)
    n_keep = n_keep + cnt_a + cnt_b

    zvec = jnp.zeros((L,), jnp.int32)
    dumpv = jnp.full((L,), DUMP, jnp.int32)

    def pad_body(j, _):
        sbuf[pl.ds(n_keep + j * L, L)] = zvec
        dbuf[pl.ds(n_keep + j * L, L)] = dumpv
        return 0

    lax.fori_loop(0, 40, pad_body, 0)
    nb = (n_keep + KB - 1) // KB
    nb4 = 4 * jnp.maximum(1, (nb + 3) // 4)

    def start_gather(t, p):
        pltpu.async_copy(xsp.at[sbuf.at[pl.ds(t * KB, KB)]], rows[p], gsem[p])

    def wait_gather(p):
        pltpu.make_async_copy(xsp.at[sbuf.at[pl.ds(0, KB)]], rows[p],
                              gsem[p]).wait()

    def start_scatter(t, p):
        pltpu.async_copy(rows[p], acc.at[dbuf.at[pl.ds(t * KB, KB)]],
                         ssem[p], add=True)

    def wait_scatter(p):
        pltpu.make_async_copy(rows[p], acc.at[dbuf.at[pl.ds(0, KB)]],
                              ssem[p]).wait()

    start_gather(jnp.int32(0), 0)
    start_gather(jnp.int32(1), 1)
    for t in range(2, 4):
        start_gather(jnp.int32(t), t)
        wait_gather(t - 2)
        start_scatter(jnp.int32(t - 2), t - 2)

    def group_body(g, _):
        for p in range(4):
            t = 4 * g + p
            wait_scatter(p)
            start_gather(t, p)
            wait_gather((p + 2) % 4)
            start_scatter(t - 2, (p + 2) % 4)
        return 0

    lax.fori_loop(1, nb4 // 4, group_body, 0)

    wait_gather(2)
    start_scatter(nb4 - 2, 2)
    wait_gather(3)
    start_scatter(nb4 - 1, 3)
    for p in range(4):
        wait_scatter(p)

    plsc.subcore_barrier()

    base = s * RPT
    pltpu.sync_copy(acc.at[pl.ds(base, RPT)],
                    out.at[c, pl.ds(base, RPT), :])


def kernel(x, edge_index, batch_size):
    x = x.astype(jnp.float32)
    ei = edge_index.astype(jnp.int32)
    start = jnp.asarray(batch_size, jnp.int32) - BATCH
    x2 = x.reshape(2 * N_NODES, H)
    startv = jnp.full((L,), start, jnp.int32)
    zrows = jnp.zeros((ZROWS, H), jnp.float32)
    planes = _agg_kernel(x2, ei, startv, zrows)
    x_head = lax.dynamic_slice_in_dim(x, start, BATCH, axis=0)
    return jnp.concatenate((x_head, planes[0], planes[1]), axis=-1)
